# baseline (speedup 1.0000x reference)
import functools

import jax
import jax.numpy as jnp
from jax import lax
from jax.experimental import pallas as pl
from jax.experimental.pallas import tpu as pltpu
from jax.experimental.pallas import tpu_sc as plsc

_INFO = plsc.get_sparse_core_info()
_NC = _INFO.num_cores
_NS = _INFO.num_subcores
_NW = _NC * _NS
_L = 16


@functools.partial(jax.jit, static_argnums=(2, 3, 4))
def _emb_lookup(words_t, table, b, h, d):
    bw = b // _NW
    nblk = bw // 128
    dhi = d // 8
    plane = (d // 8) * (b // 128) * 8 * 128
    dh_pitch = (b // 128) * 8 * 128
    tsz = dhi * nblk * 8 * 128
    mesh = plsc.VectorSubcoreMesh(core_axis_name="c", subcore_axis_name="s")

    @functools.partial(
        pl.kernel,
        out_type=jax.ShapeDtypeStruct((h, plane), jnp.float32),
        mesh=mesh,
        scratch_types=[
            pltpu.VMEM((2, bw), jnp.int32),
            pltpu.VMEM((2, bw, d), jnp.float32),
            pltpu.VMEM((2, tsz), jnp.float32),
            pltpu.SemaphoreType.DMA((2,)),
            pltpu.SemaphoreType.DMA((2,)),
            pltpu.SemaphoreType.DMA((2,)),
        ],
        compiler_params=pltpu.CompilerParams(
            use_tc_tiling_on_sc=False, needs_layout_passes=False),
    )
    def k(words_hbm, table_hbm, out_hbm, idx_v, rows_v, tile_v,
          sem_i, sem_g, sem_o):
        wid = lax.axis_index("s") * _NC + lax.axis_index("c")
        b0 = wid * bw
        blk0 = wid * nblk

        def start_idx(hh, s):
            pltpu.async_copy(
                words_hbm.at[hh, pl.ds(b0, bw)], idx_v.at[s], sem_i.at[s])

        def wait_idx(s):
            pltpu.make_async_copy(
                words_hbm.at[0, pl.ds(b0, bw)], idx_v.at[s],
                sem_i.at[s]).wait()

        def start_gather(s):
            pltpu.async_copy(
                table_hbm.at[idx_v.at[s]], rows_v.at[s], sem_g.at[s])

        def wait_gather(s):
            pltpu.make_async_copy(
                table_hbm.at[idx_v.at[s]], rows_v.at[s], sem_g.at[s]).wait()

        def start_out(hh, s):
            for dh in range(dhi):
                pltpu.async_copy(
                    tile_v.at[s, pl.ds(dh * nblk * 8 * 128, nblk * 8 * 128)],
                    out_hbm.at[hh, pl.ds(dh * dh_pitch + blk0 * 8 * 128,
                                         nblk * 8 * 128)],
                    sem_o.at[s])

        def wait_out(s):
            for dh in range(dhi):
                pltpu.make_async_copy(
                    tile_v.at[s, pl.ds(dh * nblk * 8 * 128, nblk * 8 * 128)],
                    out_hbm.at[0, pl.ds(dh * dh_pitch,
                                        nblk * 8 * 128)],
                    sem_o.at[s]).wait()

        iota = lax.iota(jnp.int32, _L)

        col_pat = {}
        woff_pat = {}
        for half in range(2):
            for kk in range(_L):
                dv = (iota + kk) % _L + half * _L
                col_pat[half, kk] = dv
                woff_pat[half, kk] = (
                    (dv // 8) * (nblk * 8 * 128) + (dv % 8) * 128 + iota)

        def transpose(s):
            def step_body(st, carry):
                row_idx = iota + st * _L
                wbase = (st // 8) * 1024 + (st - (st // 8) * 8) * _L
                for half in range(2):
                    for kg in range(0, _L, 8):
                        vecs = [
                            plsc.load_gather(
                                rows_v.at[s],
                                [row_idx, col_pat[half, kg + j]])
                            for j in range(8)
                        ]
                        for j in range(8):
                            plsc.store_scatter(
                                tile_v.at[s],
                                [woff_pat[half, kg + j] + wbase], vecs[j])
                return carry
            lax.fori_loop(0, bw // _L, step_body, 0)

        start_idx(0, 0)
        start_idx(1, 1)
        wait_idx(0)
        start_gather(0)

        def body(i, carry):
            for s in range(2):
                hh = 2 * i + s
                o = 1 - s

                @pl.when(hh + 1 < h)
                def _():
                    wait_idx(o)
                    start_gather(o)

                wait_gather(s)

                @pl.when(hh + 2 < h)
                def _():
                    start_idx(hh + 2, s)

                @pl.when(hh >= 2)
                def _():
                    wait_out(s)

                transpose(s)
                start_out(hh, s)
            return carry

        lax.fori_loop(0, h // 2, body, 0)
        wait_out(0)
        wait_out(1)

    return k(words_t, table)


def kernel(words, table):
    b, h = words.shape
    v, d = table.shape
    words_t = words.T
    out2 = _emb_lookup(words_t, table, b, h, d)
    out5 = out2.reshape(h, d // 8, b // 128, 8, 128)
    return out5.transpose(2, 4, 0, 1, 3).reshape(b, h, d)


# speedup vs baseline: 1.0602x; 1.0602x over previous
import functools

import jax
import jax.numpy as jnp
from jax import lax
from jax.experimental import pallas as pl
from jax.experimental.pallas import tpu as pltpu
from jax.experimental.pallas import tpu_sc as plsc

_INFO = plsc.get_sparse_core_info()
_NC = _INFO.num_cores
_NS = _INFO.num_subcores
_NW = _NC * _NS
_L = 16


@functools.partial(jax.jit, static_argnums=(2, 3, 4))
def _emb_lookup(words_t, table, b, h, d):
    bw = b // _NW
    nblk = bw // 128
    dhi = d // 8
    plane = (d // 8) * (b // 128) * 8 * 128
    dh_pitch = (b // 128) * 8 * 128
    tsz = dhi * nblk * 8 * 128
    mesh = plsc.VectorSubcoreMesh(core_axis_name="c", subcore_axis_name="s")

    @functools.partial(
        pl.kernel,
        out_type=jax.ShapeDtypeStruct((h, plane), jnp.float32),
        mesh=mesh,
        scratch_types=[
            pltpu.VMEM((4, bw), jnp.int32),
            pltpu.VMEM((4, bw, d), jnp.float32),
            pltpu.VMEM((2, tsz), jnp.float32),
            pltpu.SemaphoreType.DMA((4,)),
            pltpu.SemaphoreType.DMA((4,)),
            pltpu.SemaphoreType.DMA((2,)),
        ],
        compiler_params=pltpu.CompilerParams(
            use_tc_tiling_on_sc=False, needs_layout_passes=False),
    )
    def k(words_hbm, table_hbm, out_hbm, idx_v, rows_v, tile_v,
          sem_i, sem_g, sem_o):
        wid = lax.axis_index("s") * _NC + lax.axis_index("c")
        b0 = wid * bw
        blk0 = wid * nblk

        def start_idx(hh, s):
            pltpu.async_copy(
                words_hbm.at[hh, pl.ds(b0, bw)], idx_v.at[s], sem_i.at[s])

        def wait_idx(s):
            pltpu.make_async_copy(
                words_hbm.at[0, pl.ds(b0, bw)], idx_v.at[s],
                sem_i.at[s]).wait()

        def start_gather(s):
            pltpu.async_copy(
                table_hbm.at[idx_v.at[s]], rows_v.at[s], sem_g.at[s])

        def wait_gather(s):
            pltpu.make_async_copy(
                table_hbm.at[idx_v.at[s]], rows_v.at[s], sem_g.at[s]).wait()

        def start_out(hh, s):
            for dh in range(dhi):
                pltpu.async_copy(
                    tile_v.at[s, pl.ds(dh * nblk * 8 * 128, nblk * 8 * 128)],
                    out_hbm.at[hh, pl.ds(dh * dh_pitch + blk0 * 8 * 128,
                                         nblk * 8 * 128)],
                    sem_o.at[s])

        def wait_out(s):
            for dh in range(dhi):
                pltpu.make_async_copy(
                    tile_v.at[s, pl.ds(dh * nblk * 8 * 128, nblk * 8 * 128)],
                    out_hbm.at[0, pl.ds(dh * dh_pitch,
                                        nblk * 8 * 128)],
                    sem_o.at[s]).wait()

        iota = lax.iota(jnp.int32, _L)

        col_pat = {}
        woff_pat = {}
        for half in range(2):
            for kk in range(_L):
                dv = (iota + kk) % _L + half * _L
                col_pat[half, kk] = dv
                woff_pat[half, kk] = (
                    (dv // 8) * (nblk * 8 * 128) + (dv % 8) * 128 + iota)

        def transpose(r, t):
            def step_body(st, carry):
                row_idx = iota + st * _L
                wbase = (st // 8) * 1024 + (st - (st // 8) * 8) * _L
                for half in range(2):
                    for kg in range(0, _L, 8):
                        vecs = [
                            plsc.load_gather(
                                rows_v.at[r],
                                [row_idx, col_pat[half, kg + j]])
                            for j in range(8)
                        ]
                        for j in range(8):
                            plsc.store_scatter(
                                tile_v.at[t],
                                [woff_pat[half, kg + j] + wbase], vecs[j])
                return carry
            lax.fori_loop(0, bw // _L, step_body, 0)

        start_idx(0, 0)
        start_idx(1, 1)
        start_idx(2, 2)
        wait_idx(0)
        start_gather(0)
        wait_idx(1)
        start_gather(1)

        def body(i, carry):
            for u in range(4):
                hh = 4 * i + u
                t = u % 2

                @pl.when(hh + 2 < h)
                def _():
                    wait_idx((u + 2) % 4)
                    start_gather((u + 2) % 4)

                @pl.when(hh + 3 < h)
                def _():
                    start_idx(hh + 3, (u + 3) % 4)

                wait_gather(u)

                @pl.when(hh >= 2)
                def _():
                    wait_out(t)

                transpose(u, t)
                start_out(hh, t)
            return carry

        lax.fori_loop(0, h // 4, body, 0)
        wait_out(0)
        wait_out(1)

    return k(words_t, table)


def kernel(words, table):
    b, h = words.shape
    v, d = table.shape
    words_t = words.T
    out2 = _emb_lookup(words_t, table, b, h, d)
    out5 = out2.reshape(h, d // 8, b // 128, 8, 128)
    return out5.transpose(2, 4, 0, 1, 3).reshape(b, h, d)


# speedup vs baseline: 1.0694x; 1.0087x over previous
import functools

import jax
import jax.numpy as jnp
from jax import lax
from jax.experimental import pallas as pl
from jax.experimental.pallas import tpu as pltpu
from jax.experimental.pallas import tpu_sc as plsc

_INFO = plsc.get_sparse_core_info()
_NC = _INFO.num_cores
_NS = _INFO.num_subcores
_NW = _NC * _NS
_L = 16
_NWB = 16
_NWH = 2


@functools.partial(jax.jit, static_argnums=(2, 3, 4))
def _emb_lookup(words_t, table, b, h, d):
    bw = b // _NWB
    hw = h // _NWH
    nblk = bw // 128
    plane = (d // 8) * (b // 128) * 8 * 128
    dh_pitch = (b // 128) * 8 * 128
    hsz = nblk * 8 * 128
    mesh = plsc.VectorSubcoreMesh(core_axis_name="c", subcore_axis_name="s")

    @functools.partial(
        pl.kernel,
        out_type=jax.ShapeDtypeStruct((h, plane), jnp.float32),
        mesh=mesh,
        scratch_types=[
            pltpu.VMEM((2, bw), jnp.int32),
            pltpu.VMEM((2, bw, d), jnp.float32),
            pltpu.VMEM((2, 2 * hsz), jnp.float32),
            pltpu.SemaphoreType.DMA((2,)),
            pltpu.SemaphoreType.DMA((2,)),
            pltpu.SemaphoreType.DMA((2,)),
        ],
        compiler_params=pltpu.CompilerParams(
            use_tc_tiling_on_sc=False, needs_layout_passes=False),
    )
    def k(words_hbm, table_hbm, out_hbm, idx_v, rows_v, tile_v,
          sem_i, sem_g, sem_o):
        wid = lax.axis_index("s") * _NC + lax.axis_index("c")
        wh = wid // _NWB
        wb = wid - wh * _NWB
        b0 = wb * bw
        h0 = wh * hw
        blk0 = wb * nblk

        def start_idx(hh, s):
            pltpu.async_copy(
                words_hbm.at[h0 + hh, pl.ds(b0, bw)], idx_v.at[s],
                sem_i.at[s])

        def wait_idx(s):
            pltpu.make_async_copy(
                words_hbm.at[0, pl.ds(b0, bw)], idx_v.at[s],
                sem_i.at[s]).wait()

        def start_gather(s):
            pltpu.async_copy(
                table_hbm.at[idx_v.at[s]], rows_v.at[s], sem_g.at[s])

        def wait_gather(s):
            pltpu.make_async_copy(
                table_hbm.at[idx_v.at[s]], rows_v.at[s], sem_g.at[s]).wait()

        def start_out(hh, half):
            for dhp in range(2):
                pltpu.async_copy(
                    tile_v.at[half, pl.ds(dhp * hsz, hsz)],
                    out_hbm.at[h0 + hh,
                               pl.ds((half * 2 + dhp) * dh_pitch
                                     + blk0 * 8 * 128, hsz)],
                    sem_o.at[half])

        def wait_out(half):
            for dhp in range(2):
                pltpu.make_async_copy(
                    tile_v.at[half, pl.ds(dhp * hsz, hsz)],
                    out_hbm.at[0, pl.ds(dhp * dh_pitch, hsz)],
                    sem_o.at[half]).wait()

        iota = lax.iota(jnp.int32, _L)

        col_pat = {}
        woff_pat = {}
        for kk in range(_L):
            dd = (iota + kk) % _L
            woff_pat[kk] = (dd // 8) * hsz + (dd % 8) * 128 + iota
            for half in range(2):
                col_pat[half, kk] = dd + half * _L

        def transpose_half(r, half):
            def step_body(st, carry):
                row_idx = iota + st * _L
                wbase = (st // 8) * 1024 + (st - (st // 8) * 8) * _L
                for kg in range(0, _L, 8):
                    vecs = [
                        plsc.load_gather(
                            rows_v.at[r],
                            [row_idx, col_pat[half, kg + j]])
                        for j in range(8)
                    ]
                    for j in range(8):
                        plsc.store_scatter(
                            tile_v.at[half],
                            [woff_pat[kg + j] + wbase], vecs[j])
                return carry
            lax.fori_loop(0, bw // _L, step_body, 0)

        start_idx(0, 0)
        start_idx(1, 1)
        wait_idx(0)
        start_gather(0)

        def body(i, carry):
            for u in range(2):
                hh = 2 * i + u
                o = 1 - u

                @pl.when(hh + 1 < hw)
                def _():
                    wait_idx(o)
                    start_gather(o)

                wait_gather(u)

                @pl.when(hh + 2 < hw)
                def _():
                    start_idx(hh + 2, u)

                for half in range(2):
                    @pl.when(hh >= 1)
                    def _():
                        wait_out(half)

                    transpose_half(u, half)
                    start_out(hh, half)
            return carry

        lax.fori_loop(0, hw // 2, body, 0)
        wait_out(0)
        wait_out(1)

    return k(words_t, table)


def kernel(words, table):
    b, h = words.shape
    v, d = table.shape
    words_t = words.T
    out2 = _emb_lookup(words_t, table, b, h, d)
    out5 = out2.reshape(h, d // 8, b // 128, 8, 128)
    return out5.transpose(2, 4, 0, 1, 3).reshape(b, h, d)
